# baseline (speedup 1.0000x reference)
import jax
import jax.numpy as jnp
from jax.experimental import pallas as pl
from jax.experimental.pallas import tpu as pltpu
from jax.scipy.special import logsumexp

_COLS = 1024
_COLS_OH = 2048


def _reduce_block(ln_ref, wstack_ref, c0_ref, g_ref, idx_ref, s_ref):
    ln = ln_ref[...]
    ws = wstack_ref[...]
    g = g_ref[...]
    K, C = g.shape

    z = ln + g
    zmax = jnp.max(z, axis=0, keepdims=True)
    e = jnp.exp(z - zmax)
    iota = jax.lax.broadcasted_iota(jnp.int32, (K, C), 0)
    idx_ref[...] = jnp.min(jnp.where(e >= 1.0, iota, K), axis=0, keepdims=True)

    eg = jnp.exp2(g * jnp.float32(-1.4426950408889634)).astype(jnp.bfloat16)
    sums = jax.lax.dot_general(
        ws, eg, (((1,), (0,)), ((), ())), preferred_element_type=jnp.float32
    )
    a = jnp.log(sums[0:1, :])
    b = jnp.log(sums[1:2, :])
    s_ref[...] = c0_ref[...] + K * (b - a)


def _onehot_block(idx_ref, d_ref):
    K, C = d_ref.shape
    iota = jax.lax.broadcasted_iota(jnp.int32, (K, C), 0)
    d_ref[...] = (iota == idx_ref[...]).astype(jnp.float32)


def kernel(num_samples, temperature, logits, prior_logits, gumbel):
    K = logits.shape[-1]
    S = gumbel.shape[0]
    ln = (logits - logsumexp(logits, axis=0, keepdims=True)).reshape(K, 1)
    w = jnp.exp(prior_logits.reshape(1, K) - ln.reshape(1, K))
    wstack = jnp.concatenate(
        [jnp.ones((1, K), jnp.float32), w, jnp.zeros((6, K), jnp.float32)], axis=0
    ).astype(jnp.bfloat16)
    c0 = (jnp.sum(ln) - jnp.sum(prior_logits)).reshape(1, 1)

    gt = gumbel.T
    idx, s = pl.pallas_call(
        _reduce_block,
        grid=(S // _COLS,),
        in_specs=[
            pl.BlockSpec((K, 1), lambda i: (0, 0)),
            pl.BlockSpec((8, K), lambda i: (0, 0)),
            pl.BlockSpec((1, 1), lambda i: (0, 0)),
            pl.BlockSpec((K, _COLS), lambda i: (0, i)),
        ],
        out_specs=[
            pl.BlockSpec((1, _COLS), lambda i: (0, i)),
            pl.BlockSpec((1, _COLS), lambda i: (0, i)),
        ],
        out_shape=[
            jax.ShapeDtypeStruct((1, S), jnp.int32),
            jax.ShapeDtypeStruct((1, S), jnp.float32),
        ],
        compiler_params=pltpu.CompilerParams(
            dimension_semantics=("parallel",),
        ),
    )(ln, wstack, c0, gt)

    Dt = pl.pallas_call(
        _onehot_block,
        grid=(S // _COLS_OH,),
        in_specs=[pl.BlockSpec((1, _COLS_OH), lambda i: (0, i))],
        out_specs=pl.BlockSpec((K, _COLS_OH), lambda i: (0, i)),
        out_shape=jax.ShapeDtypeStruct((K, S), jnp.float32),
        compiler_params=pltpu.CompilerParams(
            dimension_semantics=("parallel",),
        ),
    )(idx)
    return (Dt.T, s.reshape(S))


# speedup vs baseline: 1.1249x; 1.1249x over previous
import jax
import jax.numpy as jnp
from jax.experimental import pallas as pl
from jax.experimental.pallas import tpu as pltpu
from jax.scipy.special import logsumexp

_COLS = 1024


def _gr_block(invt_ref, ln_ref, wstack_ref, c0_ref, g_ref, d_ref, s_ref):
    ln = ln_ref[...]
    ws = wstack_ref[...]
    g = g_ref[...]
    K, C = g.shape

    z = ln + g
    zmax = jnp.max(z, axis=0, keepdims=True)
    iota = jax.lax.broadcasted_iota(jnp.int32, (K, C), 0)
    idx = jnp.min(jnp.where(z >= zmax, iota, K), axis=0, keepdims=True)
    d_ref[...] = (iota == idx).astype(jnp.float32)

    eg = jnp.exp2(g * jnp.float32(-1.4426950408889634)).astype(jnp.bfloat16)
    sums = jax.lax.dot_general(
        ws, eg, (((1,), (0,)), ((), ())), preferred_element_type=jnp.float32
    )
    a = jnp.log(sums[0:1, :])
    b = jnp.log(sums[1:2, :])
    s_ref[...] = c0_ref[...] + K * (b - a)


def kernel(num_samples, temperature, logits, prior_logits, gumbel):
    K = logits.shape[-1]
    S = gumbel.shape[0]
    ln = (logits - logsumexp(logits, axis=0, keepdims=True)).reshape(K, 1)
    w = jnp.exp(prior_logits.reshape(1, K) - ln.reshape(1, K))
    wstack = jnp.concatenate(
        [jnp.ones((1, K), jnp.float32), w, jnp.zeros((6, K), jnp.float32)], axis=0
    ).astype(jnp.bfloat16)
    c0 = (jnp.sum(ln) - jnp.sum(prior_logits)).reshape(1, 1)
    invt = (1.0 / temperature).astype(jnp.float32).reshape(1, 1)

    gt = gumbel.T
    grid = S // _COLS
    Dt, s = pl.pallas_call(
        _gr_block,
        grid=(grid,),
        in_specs=[
            pl.BlockSpec((1, 1), lambda i: (0, 0)),
            pl.BlockSpec((K, 1), lambda i: (0, 0)),
            pl.BlockSpec((8, K), lambda i: (0, 0)),
            pl.BlockSpec((1, 1), lambda i: (0, 0)),
            pl.BlockSpec((K, _COLS), lambda i: (0, i)),
        ],
        out_specs=[
            pl.BlockSpec((K, _COLS), lambda i: (0, i)),
            pl.BlockSpec((1, _COLS), lambda i: (0, i)),
        ],
        out_shape=[
            jax.ShapeDtypeStruct((K, S), jnp.float32),
            jax.ShapeDtypeStruct((1, S), jnp.float32),
        ],
        compiler_params=pltpu.CompilerParams(
            dimension_semantics=("parallel",),
        ),
    )(invt, ln, wstack, c0, gt)
    return (Dt.T, s.reshape(S))


# speedup vs baseline: 1.1830x; 1.0517x over previous
import jax
import jax.numpy as jnp
from jax.experimental import pallas as pl
from jax.experimental.pallas import tpu as pltpu
from jax.scipy.special import logsumexp

_COLS = 2048


def _gr_block(invt_ref, ln_ref, wstack_ref, c0_ref, g_ref, d_ref, s_ref):
    ln = ln_ref[...]
    ws = wstack_ref[...]
    g = g_ref[...]
    K, C = g.shape

    z = ln + g
    zmax = jnp.max(z, axis=0, keepdims=True)
    iota = jax.lax.broadcasted_iota(jnp.int32, (K, C), 0)
    idx = jnp.min(jnp.where(z >= zmax, iota, K), axis=0, keepdims=True)
    d_ref[...] = (iota == idx).astype(jnp.float32)

    eg = jnp.exp2(g * jnp.float32(-1.4426950408889634)).astype(jnp.bfloat16)
    sums = jax.lax.dot_general(
        ws, eg, (((1,), (0,)), ((), ())), preferred_element_type=jnp.float32
    )
    a = jnp.log(sums[0:1, :])
    b = jnp.log(sums[1:2, :])
    s_ref[...] = c0_ref[...] + K * (b - a)


def kernel(num_samples, temperature, logits, prior_logits, gumbel):
    K = logits.shape[-1]
    S = gumbel.shape[0]
    ln = (logits - logsumexp(logits, axis=0, keepdims=True)).reshape(K, 1)
    w = jnp.exp(prior_logits.reshape(1, K) - ln.reshape(1, K))
    wstack = jnp.concatenate(
        [jnp.ones((1, K), jnp.float32), w, jnp.zeros((6, K), jnp.float32)], axis=0
    ).astype(jnp.bfloat16)
    c0 = (jnp.sum(ln) - jnp.sum(prior_logits)).reshape(1, 1)
    invt = (1.0 / temperature).astype(jnp.float32).reshape(1, 1)

    gt = gumbel.T
    grid = S // _COLS
    Dt, s = pl.pallas_call(
        _gr_block,
        grid=(grid,),
        in_specs=[
            pl.BlockSpec((1, 1), lambda i: (0, 0)),
            pl.BlockSpec((K, 1), lambda i: (0, 0)),
            pl.BlockSpec((8, K), lambda i: (0, 0)),
            pl.BlockSpec((1, 1), lambda i: (0, 0)),
            pl.BlockSpec((K, _COLS), lambda i: (0, i)),
        ],
        out_specs=[
            pl.BlockSpec((K, _COLS), lambda i: (0, i)),
            pl.BlockSpec((1, _COLS), lambda i: (0, i)),
        ],
        out_shape=[
            jax.ShapeDtypeStruct((K, S), jnp.float32),
            jax.ShapeDtypeStruct((1, S), jnp.float32),
        ],
        compiler_params=pltpu.CompilerParams(
            dimension_semantics=("parallel",),
        ),
    )(invt, ln, wstack, c0, gt)
    return (Dt.T, s.reshape(S))


# speedup vs baseline: 1.1959x; 1.0108x over previous
import jax
import jax.numpy as jnp
from jax.experimental import pallas as pl
from jax.experimental.pallas import tpu as pltpu
from jax.scipy.special import logsumexp

_COLS = 2048


def _gr_block(ln_ref, wstack_ref, c0_ref, g_ref, d_ref, s_ref):
    ln = ln_ref[...]
    ws = wstack_ref[...]
    g = g_ref[...]
    K, C = g.shape

    z = ln + g
    zmax = jnp.max(z, axis=0, keepdims=True)
    iota = jax.lax.broadcasted_iota(jnp.int32, (K, C), 0)
    idx = jnp.min(jnp.where(z >= zmax, iota, K), axis=0, keepdims=True)
    d_ref[...] = (iota == idx).astype(jnp.float32)

    eg = jnp.exp2(g * jnp.float32(-1.4426950408889634)).astype(jnp.bfloat16)
    sums = jax.lax.dot_general(
        ws, eg, (((1,), (0,)), ((), ())), preferred_element_type=jnp.float32
    )
    a = jnp.log(sums[0:1, :])
    b = jnp.log(sums[1:2, :])
    s_ref[...] = c0_ref[...] + K * (b - a)


def kernel(num_samples, temperature, logits, prior_logits, gumbel):
    K = logits.shape[-1]
    S = gumbel.shape[0]
    ln = (logits - logsumexp(logits, axis=0, keepdims=True)).reshape(K, 1)
    w = jnp.exp(prior_logits.reshape(1, K) - ln.reshape(1, K))
    wstack = jnp.concatenate(
        [jnp.ones((1, K), jnp.float32), w, jnp.zeros((6, K), jnp.float32)], axis=0
    ).astype(jnp.bfloat16)
    c0 = (jnp.sum(ln) - jnp.sum(prior_logits)).reshape(1, 1)

    gt = gumbel.T
    grid = S // _COLS
    Dt, s = pl.pallas_call(
        _gr_block,
        grid=(grid,),
        in_specs=[
            pl.BlockSpec((K, 1), lambda i: (0, 0)),
            pl.BlockSpec((8, K), lambda i: (0, 0)),
            pl.BlockSpec((1, 1), lambda i: (0, 0)),
            pl.BlockSpec((K, _COLS), lambda i: (0, i)),
        ],
        out_specs=[
            pl.BlockSpec((K, _COLS), lambda i: (0, i)),
            pl.BlockSpec((1, _COLS), lambda i: (0, i)),
        ],
        out_shape=[
            jax.ShapeDtypeStruct((K, S), jnp.float32),
            jax.ShapeDtypeStruct((1, S), jnp.float32),
        ],
        compiler_params=pltpu.CompilerParams(
            dimension_semantics=("parallel",),
        ),
    )(ln, wstack, c0, gt)
    return (Dt.T, s.reshape(S))
